# Initial kernel scaffold; baseline (speedup 1.0000x reference)
import jax
import jax.numpy as jnp
from jax.experimental import pallas as pl


def kernel(xd, xm, edge_index, edge_index_test, W_mlp1, b_mlp1, W_mlp2, b_mlp2, W_gcn, b_gcn):
    raise NotImplementedError("write your pallas kernel here")


# speedup vs baseline: 5.8786x; 5.8786x over previous
import dataclasses
import functools

import jax
import jax.numpy as jnp
from jax import lax
from jax.experimental import pallas as pl
from jax.experimental.pallas import tpu as pltpu
from jax.experimental.pallas import tpu_sc as plsc

D = 128
NC, NS = 2, 16
NW = NC * NS
CH = 128
ROW_BLK = 1000


def _mlp_body(x_ref, w1_ref, b1_ref, w2_ref, b2_ref, wg_ref, out_ref):
    i = pl.program_id(0)
    use_d = i < (pl.num_programs(0) // 2)
    w = jnp.where(use_d, w1_ref[...], w2_ref[...])
    b = jnp.where(use_d, b1_ref[...], b2_ref[...])
    h = jnp.maximum(
        jnp.dot(x_ref[...], w, precision=lax.Precision.HIGHEST) + b, 0.0)
    out_ref[...] = jnp.dot(h, wg_ref[...], precision=lax.Precision.HIGHEST)


def _scale_body(xw_ref, dp_ref, out_ref):
    deg = dp_ref[0, :, 0:1] + dp_ref[1, :, 0:1] + 1.0
    out_ref[...] = xw_ref[...] * lax.rsqrt(deg)


def _final_body(mp_ref, xws_ref, dp_ref, bg_ref, out_ref):
    deg = dp_ref[0, :, 0:1] + dp_ref[1, :, 0:1] + 1.0
    dinv = lax.rsqrt(deg)
    s = mp_ref[0] + mp_ref[1] + xws_ref[...]
    out_ref[...] = jnp.maximum(dinv * s + bg_ref[...], 0.0)


def kernel(xd, xm, edge_index, edge_index_test,
           W_mlp1, b_mlp1, W_mlp2, b_mlp2, W_gcn, b_gcn):
    n_d = xd.shape[0]
    n = n_d + xm.shape[0]
    e = edge_index.shape[1]
    e_test = edge_index_test.shape[1]

    acc_rows = ((n + CH - 1) // CH + 1) * CH
    rpt = acc_rows // NS
    e_ch = -(-e // (NW * CH))
    t_ch = -(-e_test // (NW * CH))

    xin = jnp.concatenate([xd, xm], axis=0)
    src = edge_index[0].astype(jnp.int32)
    dst = edge_index[1].astype(jnp.int32)
    pad_e = NW * e_ch * CH - e
    srcc = jnp.concatenate([src, jnp.zeros((pad_e,), jnp.int32)])
    srcc = srcc.reshape(NW, e_ch, CH)
    dstc = jnp.concatenate([dst, jnp.full((pad_e,), n, jnp.int32)])
    dstc = dstc.reshape(NW, e_ch, CH)
    t0 = edge_index_test[0].astype(jnp.int32)
    t1 = edge_index_test[1].astype(jnp.int32)
    pad_t = NW * t_ch * CH - e_test
    t0c = jnp.concatenate([t0, jnp.zeros((pad_t,), jnp.int32)])
    t0c = t0c.reshape(NW, t_ch, CH)
    t1c = jnp.concatenate([t1, jnp.zeros((pad_t,), jnp.int32)])
    t1c = t1c.reshape(NW, t_ch, CH)
    ones128 = jnp.ones((CH, D), jnp.float32)
    zrow = jnp.zeros((rpt, D), jnp.float32)
    b1r = b_mlp1.reshape(1, D)
    b2r = b_mlp2.reshape(1, D)
    bgr = b_gcn.reshape(1, D)

    n_blk = n // ROW_BLK
    xw = pl.pallas_call(
        _mlp_body,
        grid=(n_blk,),
        in_specs=[
            pl.BlockSpec((ROW_BLK, D), lambda i: (i, 0)),
            pl.BlockSpec((D, D), lambda i: (0, 0)),
            pl.BlockSpec((1, D), lambda i: (0, 0)),
            pl.BlockSpec((D, D), lambda i: (0, 0)),
            pl.BlockSpec((1, D), lambda i: (0, 0)),
            pl.BlockSpec((D, D), lambda i: (0, 0)),
        ],
        out_specs=pl.BlockSpec((ROW_BLK, D), lambda i: (i, 0)),
        out_shape=jax.ShapeDtypeStruct((n, D), jnp.float32),
    )(xin, W_mlp1, b1r, W_mlp2, b2r, W_gcn)

    mesh = plsc.VectorSubcoreMesh(
        core_axis_name="c", subcore_axis_name="s",
        num_cores=NC, num_subcores=NS)

    @functools.partial(
        pl.kernel,
        out_type=jax.ShapeDtypeStruct((NC, acc_rows, D), jnp.float32),
        mesh=mesh,
        scratch_types=[
            pltpu.VMEM((CH,), jnp.int32),
            pltpu.VMEM((CH, D), jnp.float32),
            pltpu.VMEM_SHARED((acc_rows, D), jnp.float32),
        ],
    )
    def sc_deg(dst_hbm, ones_hbm, zdeg_hbm, degp_hbm, didx, onesv, acc):
        c = lax.axis_index("c")
        s = lax.axis_index("s")
        wid = c * NS + s
        pltpu.sync_copy(zdeg_hbm, acc.at[pl.ds(s * rpt, rpt)])
        pltpu.sync_copy(ones_hbm, onesv)
        plsc.subcore_barrier()

        @pl.loop(0, e_ch)
        def _(ch):
            pltpu.sync_copy(dst_hbm.at[wid, ch], didx)
            pltpu.sync_copy(onesv, acc.at[didx], add=True)

        plsc.subcore_barrier()
        pltpu.sync_copy(acc.at[pl.ds(s * rpt, rpt)],
                        degp_hbm.at[c, pl.ds(s * rpt, rpt)])

    degp = sc_deg(dstc, ones128, zrow)

    dp_spec = pl.BlockSpec((NC, ROW_BLK, D), lambda i: (0, i, 0))
    xws = pl.pallas_call(
        _scale_body,
        grid=(n_blk,),
        in_specs=[pl.BlockSpec((ROW_BLK, D), lambda i: (i, 0)), dp_spec],
        out_specs=pl.BlockSpec((ROW_BLK, D), lambda i: (i, 0)),
        out_shape=jax.ShapeDtypeStruct((n, D), jnp.float32),
    )(xw, degp)

    @functools.partial(
        pl.kernel,
        out_type=jax.ShapeDtypeStruct((NC, acc_rows, D), jnp.float32),
        mesh=mesh,
        scratch_types=[
            pltpu.VMEM((CH,), jnp.int32),
            pltpu.VMEM((CH,), jnp.int32),
            pltpu.VMEM((CH, D), jnp.float32),
            pltpu.VMEM_SHARED((acc_rows, D), jnp.float32),
            pltpu.SemaphoreType.DMA,
        ],
    )
    def sc_msg(src_hbm, dst_hbm, xws_hbm, zrow_hbm, msgp_hbm,
               sidx, didx, rows, acc, sem):
        c = lax.axis_index("c")
        s = lax.axis_index("s")
        wid = c * NS + s
        pltpu.sync_copy(zrow_hbm, acc.at[pl.ds(s * rpt, rpt)])
        plsc.subcore_barrier()

        @pl.loop(0, e_ch)
        def _(ch):
            pltpu.sync_copy(src_hbm.at[wid, ch], sidx)
            pltpu.sync_copy(dst_hbm.at[wid, ch], didx)
            pltpu.async_copy(xws_hbm.at[sidx], rows, sem).wait()
            pltpu.sync_copy(rows, acc.at[didx], add=True)

        plsc.subcore_barrier()
        pltpu.sync_copy(acc.at[pl.ds(s * rpt, rpt)],
                        msgp_hbm.at[c, pl.ds(s * rpt, rpt)])

    msgp = sc_msg(srcc, dstc, xws, zrow)

    xf = pl.pallas_call(
        _final_body,
        grid=(n_blk,),
        in_specs=[
            pl.BlockSpec((NC, ROW_BLK, D), lambda i: (0, i, 0)),
            pl.BlockSpec((ROW_BLK, D), lambda i: (i, 0)),
            dp_spec,
            pl.BlockSpec((1, D), lambda i: (0, 0)),
        ],
        out_specs=pl.BlockSpec((ROW_BLK, D), lambda i: (i, 0)),
        out_shape=jax.ShapeDtypeStruct((n, D), jnp.float32),
    )(msgp, xws, degp, bgr)

    n_grp = CH // 16
    cp = pltpu.CompilerParams()
    if "needs_layout_passes" in pltpu.CompilerParams.__dataclass_fields__:
        cp = dataclasses.replace(cp, needs_layout_passes=False)

    @functools.partial(
        pl.kernel,
        out_type=jax.ShapeDtypeStruct((NW, t_ch, CH), jnp.float32),
        mesh=mesh,
        compiler_params=cp,
        scratch_types=[
            pltpu.VMEM((CH,), jnp.int32),
            pltpu.VMEM((CH,), jnp.int32),
            pltpu.VMEM((CH, D), jnp.float32),
            pltpu.VMEM((CH, D), jnp.float32),
            pltpu.VMEM((CH,), jnp.float32),
            pltpu.SemaphoreType.DMA,
            pltpu.SemaphoreType.DMA,
        ],
    )
    def sc_score(t0_hbm, t1_hbm, xf_hbm, out_hbm,
                 t0v, t1v, ub, vb, sb, sem0, sem1):
        c = lax.axis_index("c")
        s = lax.axis_index("s")
        wid = c * NS + s
        lane = lax.iota(jnp.int32, 16)
        row_ids = [lane + g * 16 for g in range(n_grp)]

        @pl.loop(0, t_ch)
        def _(ch):
            pltpu.sync_copy(t0_hbm.at[wid, ch], t0v)
            pltpu.sync_copy(t1_hbm.at[wid, ch], t1v)
            cp0 = pltpu.async_copy(xf_hbm.at[t0v], ub, sem0)
            cp1 = pltpu.async_copy(xf_hbm.at[t1v], vb, sem1)
            cp0.wait()
            cp1.wait()

            def body(f, accs):
                col = jnp.broadcast_to(f, (16,)).astype(jnp.int32)
                new = []
                for g in range(n_grp):
                    pu = plsc.load_gather(ub, [row_ids[g], col])
                    pv = plsc.load_gather(vb, [row_ids[g], col])
                    new.append(accs[g] + pu * pv)
                return tuple(new)

            accs = lax.fori_loop(
                0, D, body, tuple(jnp.zeros((16,), jnp.float32)
                                  for _ in range(n_grp)))
            for g in range(n_grp):
                sb[pl.ds(g * 16, 16)] = accs[g]
            pltpu.sync_copy(sb, out_hbm.at[wid, ch])

    sco = sc_score(t0c, t1c, xf)
    return sco.reshape(-1)[:e_test]


# speedup vs baseline: 6.1456x; 1.0454x over previous
import dataclasses
import functools

import jax
import jax.numpy as jnp
from jax import lax
from jax.experimental import pallas as pl
from jax.experimental.pallas import tpu as pltpu
from jax.experimental.pallas import tpu_sc as plsc

D = 128
NC, NS = 2, 16
NW = NC * NS
CH = 128
ROW_BLK = 1000


def _mlp_body(x_ref, w1_ref, b1_ref, w2_ref, b2_ref, wg_ref, out_ref):
    i = pl.program_id(0)
    use_d = i < (pl.num_programs(0) // 2)
    w = jnp.where(use_d, w1_ref[...], w2_ref[...])
    b = jnp.where(use_d, b1_ref[...], b2_ref[...])
    h = jnp.maximum(
        jnp.dot(x_ref[...], w, precision=lax.Precision.HIGHEST) + b, 0.0)
    out_ref[...] = jnp.dot(h, wg_ref[...], precision=lax.Precision.HIGHEST)


def _scale_body(xw_ref, dp_ref, out_ref):
    deg = dp_ref[0, :, 0:1] + dp_ref[1, :, 0:1] + 1.0
    out_ref[...] = xw_ref[...] * lax.rsqrt(deg)


def _final_body(mp_ref, xws_ref, dp_ref, bg_ref, out_ref):
    deg = dp_ref[0, :, 0:1] + dp_ref[1, :, 0:1] + 1.0
    dinv = lax.rsqrt(deg)
    s = mp_ref[0] + mp_ref[1] + xws_ref[...]
    out_ref[...] = jnp.maximum(dinv * s + bg_ref[...], 0.0)


def kernel(xd, xm, edge_index, edge_index_test,
           W_mlp1, b_mlp1, W_mlp2, b_mlp2, W_gcn, b_gcn):
    n_d = xd.shape[0]
    n = n_d + xm.shape[0]
    e = edge_index.shape[1]
    e_test = edge_index_test.shape[1]

    acc_rows = ((n + CH - 1) // CH + 1) * CH
    rpt = acc_rows // NS
    e_ch = -(-e // (NW * CH))
    e_ch += e_ch % 2
    t_ch = -(-e_test // (NW * CH))
    t_ch += t_ch % 2

    xin = jnp.concatenate([xd, xm], axis=0)
    src = edge_index[0].astype(jnp.int32)
    dst = edge_index[1].astype(jnp.int32)
    pad_e = NW * e_ch * CH - e
    srcc = jnp.concatenate([src, jnp.zeros((pad_e,), jnp.int32)])
    srcc = srcc.reshape(NW, e_ch, CH)
    dstc = jnp.concatenate([dst, jnp.full((pad_e,), n, jnp.int32)])
    dstc = dstc.reshape(NW, e_ch, CH)
    t0 = edge_index_test[0].astype(jnp.int32)
    t1 = edge_index_test[1].astype(jnp.int32)
    pad_t = NW * t_ch * CH - e_test
    t0c = jnp.concatenate([t0, jnp.zeros((pad_t,), jnp.int32)])
    t0c = t0c.reshape(NW, t_ch, CH)
    t1c = jnp.concatenate([t1, jnp.zeros((pad_t,), jnp.int32)])
    t1c = t1c.reshape(NW, t_ch, CH)
    ones128 = jnp.ones((CH, D), jnp.float32)
    zrow = jnp.zeros((rpt, D), jnp.float32)
    b1r = b_mlp1.reshape(1, D)
    b2r = b_mlp2.reshape(1, D)
    bgr = b_gcn.reshape(1, D)

    n_blk = n // ROW_BLK
    xw = pl.pallas_call(
        _mlp_body,
        grid=(n_blk,),
        in_specs=[
            pl.BlockSpec((ROW_BLK, D), lambda i: (i, 0)),
            pl.BlockSpec((D, D), lambda i: (0, 0)),
            pl.BlockSpec((1, D), lambda i: (0, 0)),
            pl.BlockSpec((D, D), lambda i: (0, 0)),
            pl.BlockSpec((1, D), lambda i: (0, 0)),
            pl.BlockSpec((D, D), lambda i: (0, 0)),
        ],
        out_specs=pl.BlockSpec((ROW_BLK, D), lambda i: (i, 0)),
        out_shape=jax.ShapeDtypeStruct((n, D), jnp.float32),
    )(xin, W_mlp1, b1r, W_mlp2, b2r, W_gcn)

    mesh = plsc.VectorSubcoreMesh(
        core_axis_name="c", subcore_axis_name="s",
        num_cores=NC, num_subcores=NS)

    @functools.partial(
        pl.kernel,
        out_type=jax.ShapeDtypeStruct((NC, acc_rows, D), jnp.float32),
        mesh=mesh,
        scratch_types=[
            pltpu.VMEM((CH,), jnp.int32),
            pltpu.VMEM((CH, D), jnp.float32),
            pltpu.VMEM_SHARED((acc_rows, D), jnp.float32),
        ],
    )
    def sc_deg(dst_hbm, ones_hbm, zdeg_hbm, degp_hbm, didx, onesv, acc):
        c = lax.axis_index("c")
        s = lax.axis_index("s")
        wid = c * NS + s
        pltpu.sync_copy(zdeg_hbm, acc.at[pl.ds(s * rpt, rpt)])
        pltpu.sync_copy(ones_hbm, onesv)
        plsc.subcore_barrier()

        @pl.loop(0, e_ch)
        def _(ch):
            pltpu.sync_copy(dst_hbm.at[wid, ch], didx)
            pltpu.sync_copy(onesv, acc.at[didx], add=True)

        plsc.subcore_barrier()
        pltpu.sync_copy(acc.at[pl.ds(s * rpt, rpt)],
                        degp_hbm.at[c, pl.ds(s * rpt, rpt)])

    degp = sc_deg(dstc, ones128, zrow)

    dp_spec = pl.BlockSpec((NC, ROW_BLK, D), lambda i: (0, i, 0))
    xws = pl.pallas_call(
        _scale_body,
        grid=(n_blk,),
        in_specs=[pl.BlockSpec((ROW_BLK, D), lambda i: (i, 0)), dp_spec],
        out_specs=pl.BlockSpec((ROW_BLK, D), lambda i: (i, 0)),
        out_shape=jax.ShapeDtypeStruct((n, D), jnp.float32),
    )(xw, degp)

    @functools.partial(
        pl.kernel,
        out_type=jax.ShapeDtypeStruct((NC, acc_rows, D), jnp.float32),
        mesh=mesh,
        scratch_types=[
            pltpu.VMEM((CH,), jnp.int32),
            pltpu.VMEM((CH,), jnp.int32),
            pltpu.VMEM((CH,), jnp.int32),
            pltpu.VMEM((CH,), jnp.int32),
            pltpu.VMEM((CH, D), jnp.float32),
            pltpu.VMEM((CH, D), jnp.float32),
            pltpu.VMEM_SHARED((acc_rows, D), jnp.float32),
            pltpu.SemaphoreType.DMA,
            pltpu.SemaphoreType.DMA,
        ],
    )
    def sc_msg(src_hbm, dst_hbm, xws_hbm, zrow_hbm, msgp_hbm,
               sidx0, didx0, sidx1, didx1, rows0, rows1, acc, sem0, sem1):
        c = lax.axis_index("c")
        s = lax.axis_index("s")
        wid = c * NS + s
        pltpu.sync_copy(zrow_hbm, acc.at[pl.ds(s * rpt, rpt)])
        plsc.subcore_barrier()

        def load_idx(ch, sidxb, didxb):
            pltpu.sync_copy(src_hbm.at[wid, ch], sidxb)
            pltpu.sync_copy(dst_hbm.at[wid, ch], didxb)

        load_idx(0, sidx0, didx0)
        pltpu.async_copy(xws_hbm.at[sidx0], rows0, sem0)

        @pl.loop(0, e_ch // 2)
        def _(g):
            ch = g * 2
            load_idx(ch + 1, sidx1, didx1)
            pltpu.async_copy(xws_hbm.at[sidx1], rows1, sem1)
            pltpu.make_async_copy(xws_hbm.at[sidx0], rows0, sem0).wait()
            pltpu.sync_copy(rows0, acc.at[didx0], add=True)

            @pl.when(ch + 2 < e_ch)
            def _():
                load_idx(ch + 2, sidx0, didx0)
                pltpu.async_copy(xws_hbm.at[sidx0], rows0, sem0)

            pltpu.make_async_copy(xws_hbm.at[sidx1], rows1, sem1).wait()
            pltpu.sync_copy(rows1, acc.at[didx1], add=True)

        plsc.subcore_barrier()
        pltpu.sync_copy(acc.at[pl.ds(s * rpt, rpt)],
                        msgp_hbm.at[c, pl.ds(s * rpt, rpt)])

    msgp = sc_msg(srcc, dstc, xws, zrow)

    xf = pl.pallas_call(
        _final_body,
        grid=(n_blk,),
        in_specs=[
            pl.BlockSpec((NC, ROW_BLK, D), lambda i: (0, i, 0)),
            pl.BlockSpec((ROW_BLK, D), lambda i: (i, 0)),
            dp_spec,
            pl.BlockSpec((1, D), lambda i: (0, 0)),
        ],
        out_specs=pl.BlockSpec((ROW_BLK, D), lambda i: (i, 0)),
        out_shape=jax.ShapeDtypeStruct((n, D), jnp.float32),
    )(msgp, xws, degp, bgr)

    n_grp = CH // 16
    cp = pltpu.CompilerParams()
    if "needs_layout_passes" in pltpu.CompilerParams.__dataclass_fields__:
        cp = dataclasses.replace(cp, needs_layout_passes=False)

    @functools.partial(
        pl.kernel,
        out_type=jax.ShapeDtypeStruct((NW, t_ch, CH), jnp.float32),
        mesh=mesh,
        compiler_params=cp,
        scratch_types=[
            pltpu.VMEM((CH,), jnp.int32),
            pltpu.VMEM((CH,), jnp.int32),
            pltpu.VMEM((CH,), jnp.int32),
            pltpu.VMEM((CH,), jnp.int32),
            pltpu.VMEM((CH, D), jnp.float32),
            pltpu.VMEM((CH, D), jnp.float32),
            pltpu.VMEM((CH, D), jnp.float32),
            pltpu.VMEM((CH, D), jnp.float32),
            pltpu.VMEM((CH,), jnp.float32),
            pltpu.SemaphoreType.DMA,
            pltpu.SemaphoreType.DMA,
            pltpu.SemaphoreType.DMA,
            pltpu.SemaphoreType.DMA,
        ],
    )
    def sc_score(t0_hbm, t1_hbm, xf_hbm, out_hbm,
                 t0v0, t1v0, t0v1, t1v1, ub0, vb0, ub1, vb1, sb,
                 us0, vs0, us1, vs1):
        c = lax.axis_index("c")
        s = lax.axis_index("s")
        wid = c * NS + s
        lane = lax.iota(jnp.int32, 16)
        row_ids = [lane + g * 16 for g in range(n_grp)]
        UN = 4

        def load_idx(ch, t0b, t1b):
            pltpu.sync_copy(t0_hbm.at[wid, ch], t0b)
            pltpu.sync_copy(t1_hbm.at[wid, ch], t1b)

        def fire(t0b, t1b, ubuf, vbuf, usem, vsem):
            pltpu.async_copy(xf_hbm.at[t0b], ubuf, usem)
            pltpu.async_copy(xf_hbm.at[t1b], vbuf, vsem)

        def drain(t0b, t1b, ubuf, vbuf, usem, vsem):
            pltpu.make_async_copy(xf_hbm.at[t0b], ubuf, usem).wait()
            pltpu.make_async_copy(xf_hbm.at[t1b], vbuf, vsem).wait()

        def compute(ch, ubuf, vbuf):
            def body(j, carry):
                accs, col = carry[:-1], carry[-1]
                new = list(accs)
                for k in range(UN):
                    ck = col + k
                    for g in range(n_grp):
                        pu = plsc.load_gather(ubuf, [row_ids[g], ck])
                        pv = plsc.load_gather(vbuf, [row_ids[g], ck])
                        new[g] = new[g] + pu * pv
                return (*new, col + UN)

            init = tuple(jnp.zeros((16,), jnp.float32)
                         for _ in range(n_grp)) + (jnp.zeros((16,), jnp.int32),)
            res = lax.fori_loop(0, D // UN, body, init)
            for g in range(n_grp):
                sb[pl.ds(g * 16, 16)] = res[g]
            pltpu.sync_copy(sb, out_hbm.at[wid, ch])

        load_idx(0, t0v0, t1v0)
        fire(t0v0, t1v0, ub0, vb0, us0, vs0)

        @pl.loop(0, t_ch // 2)
        def _(g):
            ch = g * 2
            load_idx(ch + 1, t0v1, t1v1)
            fire(t0v1, t1v1, ub1, vb1, us1, vs1)
            drain(t0v0, t1v0, ub0, vb0, us0, vs0)
            compute(ch, ub0, vb0)

            @pl.when(ch + 2 < t_ch)
            def _():
                load_idx(ch + 2, t0v0, t1v0)
                fire(t0v0, t1v0, ub0, vb0, us0, vs0)

            drain(t0v1, t1v1, ub1, vb1, us1, vs1)
            compute(ch + 1, ub1, vb1)

    sco = sc_score(t0c, t1c, xf)
    return sco.reshape(-1)[:e_test]
